# baseline (speedup 1.0000x reference)
import functools

import jax
import jax.numpy as jnp
from jax import lax
from jax.experimental import pallas as pl
from jax.experimental.pallas import tpu as pltpu

N_POINTS = 10000
NUM_FEATURES = 128
NUM_CLASSES = 16
K_NN = 16
EPS_BN = 1e-5


def _head_body(x_ref, w1_ref, b1_ref, g1_ref, be1_ref, w2_ref, b2_ref, o_ref):
    x = x_ref[...]
    h = lax.dot_general(x, w1_ref[...], (((1,), (1,)), ((), ())),
                        preferred_element_type=jnp.float32) + b1_ref[...]
    m = jnp.mean(h, axis=0)
    v = jnp.mean((h - m) ** 2, axis=0)
    h = (h - m) / jnp.sqrt(v + EPS_BN) * g1_ref[...] + be1_ref[...]
    h = jnp.maximum(h, 0.0)
    o = lax.dot_general(h, w2_ref[...], (((1,), (1,)), ((), ())),
                        preferred_element_type=jnp.float32) + b2_ref[...]
    shifted = o - jnp.max(o, axis=-1, keepdims=True)
    o_ref[...] = shifted - jnp.log(jnp.sum(jnp.exp(shifted), axis=-1, keepdims=True))


def _head_pallas(xcat, p1, p2):
    n = xcat.shape[0]
    return pl.pallas_call(
        _head_body,
        out_shape=jax.ShapeDtypeStruct((n, NUM_CLASSES), jnp.float32),
    )(xcat, p1['w'], p1['b'], p1['g'], p1['be'], p2['w'], p2['b'])


def _batchnorm(x, g, b):
    m = jnp.mean(x, axis=0)
    v = jnp.var(x, axis=0)
    return (x - m) / jnp.sqrt(v + EPS_BN) * g + b


def _knn_graph(pos, k, chunk=2500):
    n = pos.shape[0]
    sq = jnp.sum(pos * pos, axis=1)
    nbrs = []
    for i in range(0, n, chunk):
        q = pos[i:i + chunk]
        d = jnp.sum(q * q, axis=1, keepdims=True) - 2.0 * (q @ pos.T) + sq[None, :]
        gidx = jnp.arange(i, i + q.shape[0])
        d = jnp.where(jnp.arange(n)[None, :] == gidx[:, None], jnp.inf, d)
        _, idx = jax.lax.top_k(-d, k)
        nbrs.append(idx)
    nbr = jnp.concatenate(nbrs, axis=0)
    row = nbr.reshape(-1)
    col = jnp.repeat(jnp.arange(n), k)
    return row, col


def _knn_query(query, keys, k, chunk=2500):
    ksq = jnp.sum(keys * keys, axis=1)
    out = []
    for i in range(0, query.shape[0], chunk):
        q = query[i:i + chunk]
        d = jnp.sum(q * q, axis=1, keepdims=True) - 2.0 * (q @ keys.T) + ksq[None, :]
        _, idx = jax.lax.top_k(-d, k)
        out.append(idx)
    return jnp.concatenate(out, axis=0)


def _edge_conv(p, x, pos, k=K_NN):
    row, col = _knn_graph(lax.stop_gradient(pos), k)
    feat = jnp.concatenate([x[row], x[col] - x[row]], axis=1)
    h = feat @ p['w1'].T + p['b1']
    h = jax.nn.relu(_batchnorm(h, p['g1'], p['be1']))
    h = h @ p['w2'].T + p['b2']
    h = jax.nn.relu(_batchnorm(h, p['g2'], p['be2']))
    agg = jnp.zeros((x.shape[0], h.shape[1]), h.dtype).at[row].max(h)
    return agg


def _fps(pos, ratio):
    n = pos.shape[0]
    m = int(n * ratio)
    pos = lax.stop_gradient(pos)

    def body(i, state):
        dist, idxs = state
        last = idxs[i - 1]
        d = jnp.sum((pos - pos[last]) ** 2, axis=1)
        dist = jnp.minimum(dist, d)
        idxs = idxs.at[i].set(jnp.argmax(dist).astype(jnp.int32))
        return dist, idxs

    dist0 = jnp.full((n,), jnp.inf, jnp.float32)
    idxs0 = jnp.zeros((m,), jnp.int32)
    _, idxs = lax.fori_loop(1, m, body, (dist0, idxs0))
    return idxs


def _knn_interpolate(x, pos_x, pos_y, k=3):
    idx = _knn_query(lax.stop_gradient(pos_y), lax.stop_gradient(pos_x), k)
    diff = pos_y[:, None, :] - pos_x[idx]
    sqd = jnp.sum(diff * diff, axis=-1)
    w = 1.0 / jnp.clip(sqd, 1e-16, None)
    w = w / jnp.sum(w, axis=1, keepdims=True)
    return jnp.sum(x[idx] * w[..., None], axis=1)


def _lbr(p, x):
    h = x @ p['w'].T + p['b']
    return jax.nn.relu(_batchnorm(h, p['g'], p['be']))


def kernel(x, pos, batch, params):
    x0, pos0 = x, pos
    x1 = _edge_conv(params['conv1'], x0, pos0)
    idx1 = _fps(pos0, 0.25)
    pos1, x1s = pos0[idx1], x1[idx1]
    x2 = _edge_conv(params['conv2'], x1s, pos1)
    idx2 = _fps(pos1, 0.25)
    pos2, x2s = pos1[idx2], x2[idx2]
    x3 = _edge_conv(params['conv3'], x2s, pos2)
    idx3 = _fps(pos2, 0.25)
    pos3, x3s = pos2[idx3], x3[idx3]
    x4 = _edge_conv(params['conv4'], x3s, pos3)
    up2 = _knn_interpolate(x4, pos3, pos2)
    d2 = _lbr(params['dec1'], jnp.concatenate([up2, x3], axis=1))
    up1 = _knn_interpolate(d2, pos2, pos1)
    d1 = _lbr(params['dec2'], jnp.concatenate([up1, x2], axis=1))
    up0 = _knn_interpolate(d1, pos1, pos0)
    d0 = _lbr(params['dec3'], jnp.concatenate([up0, x1], axis=1))
    final = jnp.concatenate([d0, x0], axis=1)
    return _head_pallas(final, params['head1'], params['head2'])


# speedup vs baseline: 2.5642x; 2.5642x over previous
import functools

import jax
import jax.numpy as jnp
from jax import lax
from jax.experimental import pallas as pl
from jax.experimental.pallas import tpu as pltpu

N_POINTS = 10000
NUM_FEATURES = 128
NUM_CLASSES = 16
K_NN = 16
EPS_BN = 1e-5


def _head_body(x_ref, w1_ref, b1_ref, g1_ref, be1_ref, w2_ref, b2_ref, o_ref):
    x = x_ref[...]
    h = lax.dot_general(x, w1_ref[...], (((1,), (1,)), ((), ())),
                        preferred_element_type=jnp.float32) + b1_ref[...]
    m = jnp.mean(h, axis=0)
    v = jnp.mean((h - m) ** 2, axis=0)
    h = (h - m) / jnp.sqrt(v + EPS_BN) * g1_ref[...] + be1_ref[...]
    h = jnp.maximum(h, 0.0)
    o = lax.dot_general(h, w2_ref[...], (((1,), (1,)), ((), ())),
                        preferred_element_type=jnp.float32) + b2_ref[...]
    shifted = o - jnp.max(o, axis=-1, keepdims=True)
    o_ref[...] = shifted - jnp.log(jnp.sum(jnp.exp(shifted), axis=-1, keepdims=True))


def _head_pallas(xcat, p1, p2):
    n = xcat.shape[0]
    return pl.pallas_call(
        _head_body,
        out_shape=jax.ShapeDtypeStruct((n, NUM_CLASSES), jnp.float32),
    )(xcat, p1['w'], p1['b'], p1['g'], p1['be'], p2['w'], p2['b'])


def _fps_body(m, n, r, planes_ref, prow_ref, out_ref):
    fio = (lax.broadcasted_iota(jnp.int32, (r, 128), 0) * 128
           + lax.broadcasted_iota(jnp.int32, (r, 128), 1))
    px = planes_ref[0]
    py = planes_ref[1]
    pz = planes_ref[2]
    dist0 = jnp.where(fio < n, jnp.inf, -jnp.inf).astype(jnp.float32)
    out_ref[pl.ds(0, 1), :] = jnp.zeros((1, 1), jnp.int32)

    def step(i, carry):
        dist, last = carry
        prow = prow_ref[pl.ds(last, 1), :]
        lx, ly, lz = prow[0, 0], prow[0, 1], prow[0, 2]
        dx, dy, dz = px - lx, py - ly, pz - lz
        d = (dx * dx + dy * dy) + dz * dz
        dist = jnp.minimum(dist, d)
        mx = jnp.max(dist)
        idx = jnp.min(jnp.where(dist == mx, fio, jnp.int32(2**30)))
        out_ref[pl.ds(i, 1), :] = jnp.full((1, 1), idx, jnp.int32)
        return dist, idx

    lax.fori_loop(1, m, step, (dist0, jnp.int32(0)), unroll=False)


def _fps_pallas(pos, ratio):
    n = pos.shape[0]
    m = int(n * ratio)
    p = ((n + 127) // 128) * 128
    r = p // 128
    planes = jnp.pad(pos, ((0, p - n), (0, 0))).T.reshape(3, r, 128)
    prow = jnp.pad(pos, ((0, p - n), (0, 125)))
    out = pl.pallas_call(
        functools.partial(_fps_body, m, n, r),
        out_shape=jax.ShapeDtypeStruct((m, 1), jnp.int32),
    )(planes, prow)
    return out[:, 0]


def _batchnorm(x, g, b):
    m = jnp.mean(x, axis=0)
    v = jnp.var(x, axis=0)
    return (x - m) / jnp.sqrt(v + EPS_BN) * g + b


def _knn_graph(pos, k, chunk=2500):
    n = pos.shape[0]
    sq = jnp.sum(pos * pos, axis=1)
    nbrs = []
    for i in range(0, n, chunk):
        q = pos[i:i + chunk]
        d = jnp.sum(q * q, axis=1, keepdims=True) - 2.0 * (q @ pos.T) + sq[None, :]
        gidx = jnp.arange(i, i + q.shape[0])
        d = jnp.where(jnp.arange(n)[None, :] == gidx[:, None], jnp.inf, d)
        _, idx = jax.lax.top_k(-d, k)
        nbrs.append(idx)
    nbr = jnp.concatenate(nbrs, axis=0)
    row = nbr.reshape(-1)
    col = jnp.repeat(jnp.arange(n), k)
    return row, col


def _knn_query(query, keys, k, chunk=2500):
    ksq = jnp.sum(keys * keys, axis=1)
    out = []
    for i in range(0, query.shape[0], chunk):
        q = query[i:i + chunk]
        d = jnp.sum(q * q, axis=1, keepdims=True) - 2.0 * (q @ keys.T) + ksq[None, :]
        _, idx = jax.lax.top_k(-d, k)
        out.append(idx)
    return jnp.concatenate(out, axis=0)


def _edge_conv(p, x, pos, k=K_NN):
    row, col = _knn_graph(lax.stop_gradient(pos), k)
    feat = jnp.concatenate([x[row], x[col] - x[row]], axis=1)
    h = feat @ p['w1'].T + p['b1']
    h = jax.nn.relu(_batchnorm(h, p['g1'], p['be1']))
    h = h @ p['w2'].T + p['b2']
    h = jax.nn.relu(_batchnorm(h, p['g2'], p['be2']))
    agg = jnp.zeros((x.shape[0], h.shape[1]), h.dtype).at[row].max(h)
    return agg


def _fps(pos, ratio):
    n = pos.shape[0]
    m = int(n * ratio)
    pos = lax.stop_gradient(pos)

    def body(i, state):
        dist, idxs = state
        last = idxs[i - 1]
        d = jnp.sum((pos - pos[last]) ** 2, axis=1)
        dist = jnp.minimum(dist, d)
        idxs = idxs.at[i].set(jnp.argmax(dist).astype(jnp.int32))
        return dist, idxs

    dist0 = jnp.full((n,), jnp.inf, jnp.float32)
    idxs0 = jnp.zeros((m,), jnp.int32)
    _, idxs = lax.fori_loop(1, m, body, (dist0, idxs0))
    return idxs


def _knn_interpolate(x, pos_x, pos_y, k=3):
    idx = _knn_query(lax.stop_gradient(pos_y), lax.stop_gradient(pos_x), k)
    diff = pos_y[:, None, :] - pos_x[idx]
    sqd = jnp.sum(diff * diff, axis=-1)
    w = 1.0 / jnp.clip(sqd, 1e-16, None)
    w = w / jnp.sum(w, axis=1, keepdims=True)
    return jnp.sum(x[idx] * w[..., None], axis=1)


def _lbr(p, x):
    h = x @ p['w'].T + p['b']
    return jax.nn.relu(_batchnorm(h, p['g'], p['be']))


def kernel(x, pos, batch, params):
    x0, pos0 = x, pos
    x1 = _edge_conv(params['conv1'], x0, pos0)
    idx1 = _fps_pallas(pos0, 0.25)
    pos1, x1s = pos0[idx1], x1[idx1]
    x2 = _edge_conv(params['conv2'], x1s, pos1)
    idx2 = _fps_pallas(pos1, 0.25)
    pos2, x2s = pos1[idx2], x2[idx2]
    x3 = _edge_conv(params['conv3'], x2s, pos2)
    idx3 = _fps_pallas(pos2, 0.25)
    pos3, x3s = pos2[idx3], x3[idx3]
    x4 = _edge_conv(params['conv4'], x3s, pos3)
    up2 = _knn_interpolate(x4, pos3, pos2)
    d2 = _lbr(params['dec1'], jnp.concatenate([up2, x3], axis=1))
    up1 = _knn_interpolate(d2, pos2, pos1)
    d1 = _lbr(params['dec2'], jnp.concatenate([up1, x2], axis=1))
    up0 = _knn_interpolate(d1, pos1, pos0)
    d0 = _lbr(params['dec3'], jnp.concatenate([up0, x1], axis=1))
    final = jnp.concatenate([d0, x0], axis=1)
    return _head_pallas(final, params['head1'], params['head2'])


# speedup vs baseline: 7.6528x; 2.9845x over previous
import functools

import jax
import jax.numpy as jnp
from jax import lax
from jax.experimental import pallas as pl
from jax.experimental.pallas import tpu as pltpu

N_POINTS = 10000
NUM_FEATURES = 128
NUM_CLASSES = 16
K_NN = 16
EPS_BN = 1e-5


def _head_body(x_ref, w1_ref, b1_ref, g1_ref, be1_ref, w2_ref, b2_ref, o_ref):
    x = x_ref[...]
    h = lax.dot_general(x, w1_ref[...], (((1,), (1,)), ((), ())),
                        preferred_element_type=jnp.float32) + b1_ref[...]
    m = jnp.mean(h, axis=0)
    v = jnp.mean((h - m) ** 2, axis=0)
    h = (h - m) / jnp.sqrt(v + EPS_BN) * g1_ref[...] + be1_ref[...]
    h = jnp.maximum(h, 0.0)
    o = lax.dot_general(h, w2_ref[...], (((1,), (1,)), ((), ())),
                        preferred_element_type=jnp.float32) + b2_ref[...]
    shifted = o - jnp.max(o, axis=-1, keepdims=True)
    o_ref[...] = shifted - jnp.log(jnp.sum(jnp.exp(shifted), axis=-1, keepdims=True))


def _head_pallas(xcat, p1, p2):
    n = xcat.shape[0]
    return pl.pallas_call(
        _head_body,
        out_shape=jax.ShapeDtypeStruct((n, NUM_CLASSES), jnp.float32),
    )(xcat, p1['w'], p1['b'], p1['g'], p1['be'], p2['w'], p2['b'])


def _fps_body(m, n, r, planes_ref, prow_ref, out_ref):
    fio = (lax.broadcasted_iota(jnp.int32, (r, 128), 0) * 128
           + lax.broadcasted_iota(jnp.int32, (r, 128), 1))
    px = planes_ref[0]
    py = planes_ref[1]
    pz = planes_ref[2]
    dist0 = jnp.where(fio < n, jnp.inf, -jnp.inf).astype(jnp.float32)
    out_ref[pl.ds(0, 1), :] = jnp.zeros((1, 1), jnp.int32)

    def step(i, carry):
        dist, last = carry
        prow = prow_ref[pl.ds(last, 1), :]
        lx, ly, lz = prow[0, 0], prow[0, 1], prow[0, 2]
        dx, dy, dz = px - lx, py - ly, pz - lz
        d = (dx * dx + dy * dy) + dz * dz
        dist = jnp.minimum(dist, d)
        mx = jnp.max(dist)
        idx = jnp.min(jnp.where(dist == mx, fio, jnp.int32(2**30)))
        out_ref[pl.ds(i, 1), :] = jnp.full((1, 1), idx, jnp.int32)
        return dist, idx

    lax.fori_loop(1, m, step, (dist0, jnp.int32(0)), unroll=False)


def _fps_pallas(pos, ratio):
    n = pos.shape[0]
    m = int(n * ratio)
    p = ((n + 127) // 128) * 128
    r = p // 128
    planes = jnp.pad(pos, ((0, p - n), (0, 0))).T.reshape(3, r, 128)
    prow = jnp.pad(pos, ((0, p - n), (0, 125)))
    out = pl.pallas_call(
        functools.partial(_fps_body, m, n, r),
        out_shape=jax.ShapeDtypeStruct((m, 1), jnp.int32),
    )(planes, prow)
    return out[:, 0]


def _knn_body(n_q, n_k, p, ch, k, excl, planes_ref, q_ref, out_ref):
    i = pl.program_id(0)
    px, py, pz = planes_ref[0], planes_ref[1], planes_ref[2]
    qx, qy, qz = q_ref[:, 0:1], q_ref[:, 1:2], q_ref[:, 2:3]
    dot = (qx * px + qy * py) + qz * pz
    qsq = (qx * qx + qy * qy) + qz * qz
    sq = (px * px + py * py) + pz * pz
    d = (qsq - 2.0 * dot) + sq
    colio = lax.broadcasted_iota(jnp.int32, (ch, p), 1)
    if excl:
        rowio = lax.broadcasted_iota(jnp.int32, (ch, p), 0) + i * ch
        d = jnp.where(colio == rowio, jnp.inf, d)
    d = jnp.where(colio >= n_k, jnp.inf, d)
    outs = []
    for _ in range(k):
        mn = jnp.min(d, axis=1, keepdims=True)
        sel = jnp.min(jnp.where(d == mn, colio, jnp.int32(2**30)), axis=1)
        outs.append(sel[:, None])
        d = jnp.where(colio == sel[:, None], jnp.inf, d)
    out_ref[...] = jnp.concatenate(outs, axis=1)


def _knn_topk(query, keys, k, exclude_self):
    n_q, n_k = query.shape[0], keys.shape[0]
    p = ((n_k + 127) // 128) * 128
    qpad = ((n_q + 7) // 8) * 8
    ch = qpad if qpad <= 512 else 512
    qpad = ((n_q + ch - 1) // ch) * ch
    planes = jnp.pad(keys, ((0, p - n_k), (0, 0))).T.reshape(3, 1, p)
    qrows = jnp.pad(query, ((0, qpad - n_q), (0, 0)))
    out = pl.pallas_call(
        functools.partial(_knn_body, n_q, n_k, p, ch, k, exclude_self),
        grid=(qpad // ch,),
        in_specs=[
            pl.BlockSpec((3, 1, p), lambda i: (0, 0, 0)),
            pl.BlockSpec((ch, 3), lambda i: (i, 0)),
        ],
        out_specs=pl.BlockSpec((ch, k), lambda i: (i, 0)),
        out_shape=jax.ShapeDtypeStruct((qpad, k), jnp.int32),
    )(planes, qrows)
    return out[:n_q]


def _batchnorm(x, g, b):
    m = jnp.mean(x, axis=0)
    v = jnp.var(x, axis=0)
    return (x - m) / jnp.sqrt(v + EPS_BN) * g + b


def _knn_graph(pos, k, chunk=2500):
    n = pos.shape[0]
    sq = jnp.sum(pos * pos, axis=1)
    nbrs = []
    for i in range(0, n, chunk):
        q = pos[i:i + chunk]
        d = jnp.sum(q * q, axis=1, keepdims=True) - 2.0 * (q @ pos.T) + sq[None, :]
        gidx = jnp.arange(i, i + q.shape[0])
        d = jnp.where(jnp.arange(n)[None, :] == gidx[:, None], jnp.inf, d)
        _, idx = jax.lax.top_k(-d, k)
        nbrs.append(idx)
    nbr = jnp.concatenate(nbrs, axis=0)
    row = nbr.reshape(-1)
    col = jnp.repeat(jnp.arange(n), k)
    return row, col


def _knn_query(query, keys, k, chunk=2500):
    ksq = jnp.sum(keys * keys, axis=1)
    out = []
    for i in range(0, query.shape[0], chunk):
        q = query[i:i + chunk]
        d = jnp.sum(q * q, axis=1, keepdims=True) - 2.0 * (q @ keys.T) + ksq[None, :]
        _, idx = jax.lax.top_k(-d, k)
        out.append(idx)
    return jnp.concatenate(out, axis=0)


def _edge_conv(p, x, pos, k=K_NN):
    nbr = _knn_topk(lax.stop_gradient(pos), lax.stop_gradient(pos), k, True)
    row = nbr.reshape(-1)
    col = jnp.repeat(jnp.arange(pos.shape[0]), k)
    feat = jnp.concatenate([x[row], x[col] - x[row]], axis=1)
    h = feat @ p['w1'].T + p['b1']
    h = jax.nn.relu(_batchnorm(h, p['g1'], p['be1']))
    h = h @ p['w2'].T + p['b2']
    h = jax.nn.relu(_batchnorm(h, p['g2'], p['be2']))
    agg = jnp.zeros((x.shape[0], h.shape[1]), h.dtype).at[row].max(h)
    return agg


def _fps(pos, ratio):
    n = pos.shape[0]
    m = int(n * ratio)
    pos = lax.stop_gradient(pos)

    def body(i, state):
        dist, idxs = state
        last = idxs[i - 1]
        d = jnp.sum((pos - pos[last]) ** 2, axis=1)
        dist = jnp.minimum(dist, d)
        idxs = idxs.at[i].set(jnp.argmax(dist).astype(jnp.int32))
        return dist, idxs

    dist0 = jnp.full((n,), jnp.inf, jnp.float32)
    idxs0 = jnp.zeros((m,), jnp.int32)
    _, idxs = lax.fori_loop(1, m, body, (dist0, idxs0))
    return idxs


def _knn_interpolate(x, pos_x, pos_y, k=3):
    idx = _knn_topk(lax.stop_gradient(pos_y), lax.stop_gradient(pos_x), k, False)
    diff = pos_y[:, None, :] - pos_x[idx]
    sqd = jnp.sum(diff * diff, axis=-1)
    w = 1.0 / jnp.clip(sqd, 1e-16, None)
    w = w / jnp.sum(w, axis=1, keepdims=True)
    return jnp.sum(x[idx] * w[..., None], axis=1)


def _lbr(p, x):
    h = x @ p['w'].T + p['b']
    return jax.nn.relu(_batchnorm(h, p['g'], p['be']))


def kernel(x, pos, batch, params):
    x0, pos0 = x, pos
    x1 = _edge_conv(params['conv1'], x0, pos0)
    idx1 = _fps_pallas(pos0, 0.25)
    pos1, x1s = pos0[idx1], x1[idx1]
    x2 = _edge_conv(params['conv2'], x1s, pos1)
    idx2 = _fps_pallas(pos1, 0.25)
    pos2, x2s = pos1[idx2], x2[idx2]
    x3 = _edge_conv(params['conv3'], x2s, pos2)
    idx3 = _fps_pallas(pos2, 0.25)
    pos3, x3s = pos2[idx3], x3[idx3]
    x4 = _edge_conv(params['conv4'], x3s, pos3)
    up2 = _knn_interpolate(x4, pos3, pos2)
    d2 = _lbr(params['dec1'], jnp.concatenate([up2, x3], axis=1))
    up1 = _knn_interpolate(d2, pos2, pos1)
    d1 = _lbr(params['dec2'], jnp.concatenate([up1, x2], axis=1))
    up0 = _knn_interpolate(d1, pos1, pos0)
    d0 = _lbr(params['dec3'], jnp.concatenate([up0, x1], axis=1))
    final = jnp.concatenate([d0, x0], axis=1)
    return _head_pallas(final, params['head1'], params['head2'])
